# Initial kernel scaffold; baseline (speedup 1.0000x reference)
import jax
import jax.numpy as jnp
from jax.experimental import pallas as pl


def kernel(targets, types, neighbors, node_features, node_trans, edge_embedding_trans, trans_weights, trans_weights_s1, trans_weights_s2):
    raise NotImplementedError("write your pallas kernel here")


# speedup vs baseline: 1.8414x; 1.8414x over previous
import functools

import jax
import jax.numpy as jnp
from jax import lax
from jax.experimental import pallas as pl
from jax.experimental.pallas import tpu as pltpu
from jax.experimental.pallas import tpu_sc as plsc

N_TARGETS = 10000
F = 128
T = 2
S = 10
D = 64
E = 16
A = 32

NC = 2
NS = 16
NW = NC * NS

BLK = 32
BLOCKS_PER_W = 10
PER_W = BLK * BLOCKS_PER_W
NP = PER_W * NW
ROWS = BLK * T * S
IDX_CHUNK = 128
NCHUNK = ROWS // IDX_CHUNK


def _sc_gather_body(tgt_hbm, nbr_hbm, nf_hbm, tf_out, nbr_out,
                    tgt_idx_v, idx_vs, tgt_rows_v, rows_v, out0_v, out1_v,
                    sem, sem2):
  wid = lax.axis_index("s") * NC + lax.axis_index("c")

  def block(blk, _):
    base = wid * PER_W + blk * BLK
    pltpu.sync_copy(tgt_hbm.at[pl.ds(base, BLK)], tgt_idx_v)
    for k in range(NCHUNK):
      pltpu.sync_copy(
          nbr_hbm.at[pl.ds(base * (T * S) + k * IDX_CHUNK, IDX_CHUNK)],
          idx_vs[k])
    cps = []
    for k in range(NCHUNK):
      cps.append(pltpu.async_copy(
          nf_hbm.at[idx_vs[k]],
          rows_v.at[pl.ds(k * IDX_CHUNK, IDX_CHUNK)], sem))
    cpt = pltpu.async_copy(nf_hbm.at[tgt_idx_v], tgt_rows_v, sem2)
    for cp in cps:
      cp.wait()
    cpt.wait()

    def reduce_one(b, _):
      for t, out_v in ((0, out0_v), (1, out1_v)):
        r0 = b * (T * S) + t * S
        for j in range(F // 16):
          sl = pl.ds(j * 16, 16)
          acc = rows_v[r0, sl]
          for s in range(1, S):
            acc = acc + rows_v[r0 + s, sl]
          out_v[b, sl] = acc * (1.0 / S)
      return _

    lax.fori_loop(0, BLK, reduce_one, None)

    pltpu.sync_copy(tgt_rows_v, tf_out.at[pl.ds(base, BLK)])
    pltpu.sync_copy(out0_v, nbr_out.at[pl.ds(base, BLK)])
    pltpu.sync_copy(out1_v, nbr_out.at[pl.ds(NP + base, BLK)])
    return _

  lax.fori_loop(0, BLOCKS_PER_W, block, None)


def _sc_gather(targets_p, nbr2, node_features):
  mesh = plsc.VectorSubcoreMesh(core_axis_name="c", subcore_axis_name="s",
                                num_cores=NC, num_subcores=NS)
  fn = functools.partial(
      pl.kernel, _sc_gather_body,
      out_type=[jax.ShapeDtypeStruct((NP, F), jnp.float32),
                jax.ShapeDtypeStruct((T * NP, F), jnp.float32)],
      mesh=mesh,
      scratch_types=[
          pltpu.VMEM((BLK,), jnp.int32),
          [pltpu.VMEM((IDX_CHUNK,), jnp.int32) for _ in range(NCHUNK)],
          pltpu.VMEM((BLK, F), jnp.float32),
          pltpu.VMEM((ROWS, F), jnp.float32),
          pltpu.VMEM((BLK, F), jnp.float32),
          pltpu.VMEM((BLK, F), jnp.float32),
          pltpu.SemaphoreType.DMA,
          pltpu.SemaphoreType.DMA,
      ],
  )()
  return fn(targets_p, nbr2, node_features)


NB = 1024


def _tc_body(tf_ref, nbr0_ref, nbr1_ref, types_ref, nt_ref, eet_ref,
             tw_ref, s1_ref, s2_ref, out_ref):
  tf = tf_ref[...]
  nbr0 = nbr0_ref[...]
  nbr1 = nbr1_ref[...]
  is0 = types_ref[...] == 0

  dot = functools.partial(jnp.dot, preferred_element_type=jnp.float32)

  ea0 = dot(nbr0, eet_ref[0])
  ea1 = dot(nbr1, eet_ref[1])

  logits = []
  for ea in (ea0, ea1):
    h0 = jnp.tanh(dot(ea, s1_ref[0]))
    h1 = jnp.tanh(dot(ea, s1_ref[1]))
    l_c0 = dot(h0, s2_ref[0])
    l_c1 = dot(h1, s2_ref[1])
    logits.append(jnp.where(is0, l_c0, l_c1))
  l0, l1 = logits
  m = jnp.maximum(l0, l1)
  e0 = jnp.exp(l0 - m)
  e1 = jnp.exp(l1 - m)
  inv = 1.0 / (e0 + e1)
  a0 = e0 * inv
  a1 = e1 * inv

  ee = a0 * ea0 + a1 * ea1
  edge0 = dot(ee, tw_ref[0])
  edge1 = dot(ee, tw_ref[1])
  edge = jnp.where(is0, edge0, edge1)

  node = dot(tf, nt_ref[...])
  last = node + edge
  norm = jnp.sqrt(jnp.sum(last * last, axis=1, keepdims=True))
  out_ref[...] = last / (norm + 1e-12)


def _tc_dense(tf, nbr_mean, types_p, node_trans, eet, tw, s1, s2):
  grid = NP // NB
  full = lambda shp: pl.BlockSpec(shp, lambda i: (0,) * len(shp))
  return pl.pallas_call(
      _tc_body,
      grid=(grid,),
      in_specs=[
          pl.BlockSpec((NB, F), lambda i: (i, 0)),
          pl.BlockSpec((NB, F), lambda i: (i, 0)),
          pl.BlockSpec((NB, F), lambda i: (NP // NB + i, 0)),
          pl.BlockSpec((NB, 1), lambda i: (i, 0)),
          full((F, D)),
          full((T, F, E)),
          full((T, E, D)),
          full((T, E, A)),
          full((T, A, 1)),
      ],
      out_specs=pl.BlockSpec((NB, D), lambda i: (i, 0)),
      out_shape=jax.ShapeDtypeStruct((NP, D), jnp.float32),
  )(tf, nbr_mean, nbr_mean, types_p, node_trans, eet, tw, s1, s2)


def kernel(targets, types, neighbors, node_features, node_trans,
           edge_embedding_trans, trans_weights, trans_weights_s1,
           trans_weights_s2):
  targets = jnp.asarray(targets, jnp.int32)
  n = targets.shape[0]
  pad = NP - n
  targets_p = jnp.pad(targets, (0, pad))
  nbr_flat = jnp.pad(jnp.asarray(neighbors, jnp.int32).reshape(n, T * S),
                     ((0, pad), (0, 0)))
  nbr1 = nbr_flat.reshape(NP * T * S)
  types_p = jnp.pad(jnp.asarray(types, jnp.int32), (0, pad)).reshape(NP, 1)

  tf, nbr_mean = _sc_gather(targets_p, nbr1, node_features)
  out = _tc_dense(tf, nbr_mean, types_p, node_trans, edge_embedding_trans,
                  trans_weights, trans_weights_s1, trans_weights_s2)
  return out[:n]


# speedup vs baseline: 2.7308x; 1.4830x over previous
import functools

import jax
import jax.numpy as jnp
from jax import lax
from jax.experimental import pallas as pl
from jax.experimental.pallas import tpu as pltpu
from jax.experimental.pallas import tpu_sc as plsc

N_TARGETS = 10000
F = 128
T = 2
S = 10
D = 64
E = 16
A = 32

NC = 2
NS = 16
NW = NC * NS

BLK = 16
NBLK = 20
PER_W = BLK * NBLK
NP = PER_W * NW
ROWS = BLK * T * S
CHUNKS = ((0, 128), (128, 128), (256, 64))


def _sc_gather_body(tgt_hbm, nbr_hbm, nf_hbm, tf_out, nbr_out,
                    tgt_idx, idxs, tgt_rows, rows, out_tgt, out0, out1,
                    isem, gsem, wsem):
  wid = lax.axis_index("s") * NC + lax.axis_index("c")

  def stage(j, p):
    base = wid * PER_W + j * BLK
    pltpu.make_async_copy(tgt_hbm.at[pl.ds(base, BLK)], tgt_idx[p],
                          isem[p]).start()
    for k, (o, c) in enumerate(CHUNKS):
      pltpu.make_async_copy(nbr_hbm.at[pl.ds(base * (T * S) + o, c)],
                            idxs[p][k], isem[p]).start()

  def drain_writes(p):
    pltpu.make_async_copy(out_tgt[p], tf_out.at[pl.ds(0, BLK)], wsem[p]).wait()
    pltpu.make_async_copy(out0[p], nbr_out.at[pl.ds(0, BLK)], wsem[p]).wait()
    pltpu.make_async_copy(out1[p], nbr_out.at[pl.ds(0, BLK)], wsem[p]).wait()

  def fire(j, p):
    pltpu.make_async_copy(tgt_hbm.at[pl.ds(0, BLK)], tgt_idx[p],
                          isem[p]).wait()
    for k, (o, c) in enumerate(CHUNKS):
      pltpu.make_async_copy(nbr_hbm.at[pl.ds(0, c)], idxs[p][k],
                            isem[p]).wait()
    pltpu.make_async_copy(nf_hbm.at[tgt_idx[p]], tgt_rows[p], gsem[p]).start()
    for k, (o, c) in enumerate(CHUNKS):
      pltpu.make_async_copy(nf_hbm.at[idxs[p][k]],
                            rows[p].at[pl.ds(o, c)], gsem[p]).start()

  def wait_gathers(p):
    pltpu.make_async_copy(nf_hbm.at[tgt_idx[p]], tgt_rows[p], gsem[p]).wait()
    for k, (o, c) in enumerate(CHUNKS):
      pltpu.make_async_copy(nf_hbm.at[idxs[p][k]],
                            rows[p].at[pl.ds(o, c)], gsem[p]).wait()

  def compute(j, p, prefetch):
    wait_gathers(p)
    prefetch()
    pl.when(j >= 2)(lambda: drain_writes(p))

    def reduce_one(b, _):
      for jj in range(F // 16):
        sl = pl.ds(jj * 16, 16)
        out_tgt[p][b, sl] = tgt_rows[p][b, sl]
      for t, out_v in ((0, out0[p]), (1, out1[p])):
        r0 = b * (T * S) + t * S
        for jj in range(F // 16):
          sl = pl.ds(jj * 16, 16)
          acc = rows[p][r0, sl]
          for s in range(1, S):
            acc = acc + rows[p][r0 + s, sl]
          out_v[b, sl] = acc * (1.0 / S)
      return _

    lax.fori_loop(0, BLK, reduce_one, None)
    base = wid * PER_W + j * BLK
    pltpu.make_async_copy(out_tgt[p], tf_out.at[pl.ds(base, BLK)],
                          wsem[p]).start()
    pltpu.make_async_copy(out0[p], nbr_out.at[pl.ds(base, BLK)],
                          wsem[p]).start()
    pltpu.make_async_copy(out1[p], nbr_out.at[pl.ds(NP + base, BLK)],
                          wsem[p]).start()

  stage(0, 0)
  fire(0, 0)
  stage(1, 1)

  def body(i, _):
    j = 2 * i
    fire(j + 1, 1)
    compute(j, 0,
            lambda: pl.when(j + 2 < NBLK)(lambda: stage(j + 2, 0)))
    j2 = j + 1
    pl.when(j2 + 1 < NBLK)(lambda: fire(j2 + 1, 0))
    compute(j2, 1,
            lambda: pl.when(j2 + 2 < NBLK)(lambda: stage(j2 + 2, 1)))
    return _

  lax.fori_loop(0, NBLK // 2, body, None)
  drain_writes(0)
  drain_writes(1)


def _sc_gather(targets_p, nbr2, node_features):
  mesh = plsc.VectorSubcoreMesh(core_axis_name="c", subcore_axis_name="s",
                                num_cores=NC, num_subcores=NS)
  pair = lambda shp, dt: [pltpu.VMEM(shp, dt) for _ in range(2)]
  fn = functools.partial(
      pl.kernel, _sc_gather_body,
      out_type=[jax.ShapeDtypeStruct((NP, F), jnp.float32),
                jax.ShapeDtypeStruct((T * NP, F), jnp.float32)],
      mesh=mesh,
      scratch_types=[
          pair((BLK,), jnp.int32),
          [[pltpu.VMEM((c,), jnp.int32) for _, c in CHUNKS]
           for _ in range(2)],
          pair((BLK, F), jnp.float32),
          pair((ROWS, F), jnp.float32),
          pair((BLK, F), jnp.float32),
          pair((BLK, F), jnp.float32),
          pair((BLK, F), jnp.float32),
          [pltpu.SemaphoreType.DMA for _ in range(2)],
          [pltpu.SemaphoreType.DMA for _ in range(2)],
          [pltpu.SemaphoreType.DMA for _ in range(2)],
      ],
  )()
  return fn(targets_p, nbr2, node_features)


NB = 1024


def _tc_body(tf_ref, nbr0_ref, nbr1_ref, types_ref, nt_ref, eet_ref,
             tw_ref, s1_ref, s2_ref, out_ref):
  tf = tf_ref[...]
  nbr0 = nbr0_ref[...]
  nbr1 = nbr1_ref[...]
  is0 = types_ref[...] == 0

  dot = functools.partial(jnp.dot, preferred_element_type=jnp.float32)

  ea0 = dot(nbr0, eet_ref[0])
  ea1 = dot(nbr1, eet_ref[1])

  logits = []
  for ea in (ea0, ea1):
    h0 = jnp.tanh(dot(ea, s1_ref[0]))
    h1 = jnp.tanh(dot(ea, s1_ref[1]))
    l_c0 = dot(h0, s2_ref[0])
    l_c1 = dot(h1, s2_ref[1])
    logits.append(jnp.where(is0, l_c0, l_c1))
  l0, l1 = logits
  m = jnp.maximum(l0, l1)
  e0 = jnp.exp(l0 - m)
  e1 = jnp.exp(l1 - m)
  inv = 1.0 / (e0 + e1)
  a0 = e0 * inv
  a1 = e1 * inv

  ee = a0 * ea0 + a1 * ea1
  edge0 = dot(ee, tw_ref[0])
  edge1 = dot(ee, tw_ref[1])
  edge = jnp.where(is0, edge0, edge1)

  node = dot(tf, nt_ref[...])
  last = node + edge
  norm = jnp.sqrt(jnp.sum(last * last, axis=1, keepdims=True))
  out_ref[...] = last / (norm + 1e-12)


def _tc_dense(tf, nbr_mean, types_p, node_trans, eet, tw, s1, s2):
  grid = NP // NB
  full = lambda shp: pl.BlockSpec(shp, lambda i: (0,) * len(shp))
  return pl.pallas_call(
      _tc_body,
      grid=(grid,),
      in_specs=[
          pl.BlockSpec((NB, F), lambda i: (i, 0)),
          pl.BlockSpec((NB, F), lambda i: (i, 0)),
          pl.BlockSpec((NB, F), lambda i: (NP // NB + i, 0)),
          pl.BlockSpec((NB, 1), lambda i: (i, 0)),
          full((F, D)),
          full((T, F, E)),
          full((T, E, D)),
          full((T, E, A)),
          full((T, A, 1)),
      ],
      out_specs=pl.BlockSpec((NB, D), lambda i: (i, 0)),
      out_shape=jax.ShapeDtypeStruct((NP, D), jnp.float32),
  )(tf, nbr_mean, nbr_mean, types_p, node_trans, eet, tw, s1, s2)


def kernel(targets, types, neighbors, node_features, node_trans,
           edge_embedding_trans, trans_weights, trans_weights_s1,
           trans_weights_s2):
  targets = jnp.asarray(targets, jnp.int32)
  n = targets.shape[0]
  pad = NP - n
  targets_p = jnp.pad(targets, (0, pad))
  nbr_flat = jnp.pad(jnp.asarray(neighbors, jnp.int32).reshape(n, T * S),
                     ((0, pad), (0, 0)))
  nbr1 = nbr_flat.reshape(NP * T * S)
  types_p = jnp.pad(jnp.asarray(types, jnp.int32), (0, pad)).reshape(NP, 1)

  tf, nbr_mean = _sc_gather(targets_p, nbr1, node_features)
  out = _tc_dense(tf, nbr_mean, types_p, node_trans, edge_embedding_trans,
                  trans_weights, trans_weights_s1, trans_weights_s2)
  return out[:n]
